# baseline (speedup 1.0000x reference)
import functools

import jax
import jax.numpy as jnp
from jax.experimental import pallas as pl

_N = 8192
_D = 64
_BM = 512


def _fused_block(adj_ref, h_ref, hblk_ref, wt_ref, b_ref, out_ref):
    adj = adj_ref[...]
    acc = jnp.dot(adj, h_ref[...], preferred_element_type=jnp.float32)
    rs = jnp.sum(adj, axis=1, keepdims=True)
    pooled = acc / rs
    cat = jnp.concatenate([hblk_ref[...], pooled], axis=1)
    out_ref[...] = (
        jnp.dot(cat, wt_ref[...], preferred_element_type=jnp.float32) + b_ref[...]
    )


@jax.jit
def kernel(h, adj, W_lg, b_lg):
    n, d = h.shape
    wt = W_lg.T
    b = b_lg.reshape(1, d)
    grid = (n // _BM,)
    return pl.pallas_call(
        _fused_block,
        grid=grid,
        in_specs=[
            pl.BlockSpec((_BM, n), lambda i: (i, 0)),
            pl.BlockSpec((n, d), lambda i: (0, 0)),
            pl.BlockSpec((_BM, d), lambda i: (i, 0)),
            pl.BlockSpec((2 * d, d), lambda i: (0, 0)),
            pl.BlockSpec((1, d), lambda i: (0, 0)),
        ],
        out_specs=pl.BlockSpec((_BM, d), lambda i: (i, 0)),
        out_shape=jax.ShapeDtypeStruct((n, d), jnp.float32),
    )(adj, h, h, wt, b)


# speedup vs baseline: 1.0351x; 1.0351x over previous
import jax
import jax.numpy as jnp
from jax.experimental import pallas as pl

_N = 8192
_D = 64
_BM = 256
_NSPLIT = 4
_KS = _N // _NSPLIT


def _fused_block(*refs):
    adj_refs = refs[:_NSPLIT]
    h_ref, hblk_ref, wt_ref, b_ref, out_ref = refs[_NSPLIT:]
    h = h_ref[...]
    acc = None
    rs = None
    for j in range(_NSPLIT):
        adj = adj_refs[j][...]
        part = jnp.dot(
            adj, h[j * _KS : (j + 1) * _KS, :], preferred_element_type=jnp.float32
        )
        ps = jnp.sum(adj, axis=1, keepdims=True)
        acc = part if acc is None else acc + part
        rs = ps if rs is None else rs + ps
    pooled = acc / rs
    cat = jnp.concatenate([hblk_ref[...], pooled], axis=1)
    out_ref[...] = (
        jnp.dot(cat, wt_ref[...], preferred_element_type=jnp.float32) + b_ref[...]
    )


@jax.jit
def kernel(h, adj, W_lg, b_lg):
    n, d = h.shape
    wt = W_lg.T
    b = b_lg.reshape(1, d)
    grid = (n // _BM,)

    def slice_spec(j):
        return pl.BlockSpec((_BM, _KS), lambda i, j=j: (i, j))

    return pl.pallas_call(
        _fused_block,
        grid=grid,
        in_specs=[slice_spec(j) for j in range(_NSPLIT)]
        + [
            pl.BlockSpec((n, d), lambda i: (0, 0)),
            pl.BlockSpec((_BM, d), lambda i: (i, 0)),
            pl.BlockSpec((2 * d, d), lambda i: (0, 0)),
            pl.BlockSpec((1, d), lambda i: (0, 0)),
        ],
        out_specs=pl.BlockSpec((_BM, d), lambda i: (i, 0)),
        out_shape=jax.ShapeDtypeStruct((n, d), jnp.float32),
    )(*([adj] * _NSPLIT), h, h, wt, b)
